# Initial kernel scaffold; baseline (speedup 1.0000x reference)
import jax
import jax.numpy as jnp
from jax.experimental import pallas as pl


def kernel(x, table):
    raise NotImplementedError("write your pallas kernel here")


# speedup vs baseline: 3.7571x; 3.7571x over previous
import functools

import jax
import jax.numpy as jnp
from jax import lax
from jax.experimental import pallas as pl
from jax.experimental.pallas import tpu as pltpu
from jax.experimental.pallas import tpu_sc as plsc

XN = 4096 * 200
D = 10
VOC = 100
NC, NS, L = 2, 16, 16
NW = NC * NS
CHUNK = XN // NW
BI = 3200
NB = CHUNK // BI
G = BI // L


def _sc_body(x_hbm, tbl_hbm, out_hbm, x_v, tbl_v, out_v):
    wid = lax.axis_index("s") * NC + lax.axis_index("c")
    base = wid * CHUNK

    pltpu.sync_copy(tbl_hbm, tbl_v)

    iota = lax.iota(jnp.int32, L)
    iota_d = iota * D

    def do_block(b, _):
        off = base + b * BI
        pltpu.sync_copy(x_hbm.at[pl.ds(off, BI)], x_v)

        def do_group(g, _):
            idx16 = x_v[pl.ds(g * L, L)]
            tb = idx16 * D
            ob = iota_d + g * (L * D)
            for d in range(D):
                v = plsc.load_gather(tbl_v, [tb + d])
                plsc.store_scatter(out_v, [ob + d], v)
            return 0

        lax.fori_loop(0, G, do_group, 0)
        pltpu.sync_copy(out_v, out_hbm.at[pl.ds(off * D, BI * D)])
        return 0

    lax.fori_loop(0, NB, do_block, 0)


@jax.jit
def _sc_lookup(x_flat, tbl_flat):
    mesh = plsc.VectorSubcoreMesh(core_axis_name="c", subcore_axis_name="s")
    f = pl.kernel(
        _sc_body,
        mesh=mesh,
        out_type=jax.ShapeDtypeStruct((XN * D,), jnp.float32),
        scratch_types=[
            pltpu.VMEM((BI,), jnp.int32),
            pltpu.VMEM((VOC * D,), jnp.float32),
            pltpu.VMEM((BI * D,), jnp.float32),
        ],
        compiler_params=pltpu.CompilerParams(needs_layout_passes=False),
    )
    return f(x_flat, tbl_flat)


def kernel(x, table):
    out_flat = _sc_lookup(x.reshape(-1), table.reshape(-1))
    return out_flat.reshape(x.shape[0], x.shape[1], D)
